# baseline (speedup 1.0000x reference)
import functools

import jax
import jax.numpy as jnp
from jax import lax
from jax.experimental import pallas as pl
from jax.experimental.pallas import tpu as pltpu
from jax.experimental.pallas import tpu_sc as plsc

N = 325
E = 5200
GC = 4
NP = 384
BM = 384


BAND = NP // 16


def _sc_build_adj_body(edge_ref, zeros_ref, at_ref, edge_v, band_v, deg_v):
    cid = lax.axis_index("c")
    sid = lax.axis_index("s")

    @pl.when(cid == 0)
    def _():
        lo = sid * BAND
        pltpu.sync_copy(edge_ref, edge_v)
        pltpu.sync_copy(zeros_ref.at[pl.ds(0, BAND)], band_v)
        pltpu.sync_copy(zeros_ref.at[0], deg_v)

        ones = jnp.full((16,), 1.0, dtype=jnp.float32)

        def _deg_body(e, carry):
            d = edge_v[1, pl.ds(e * 16, 16)]
            plsc.addupdate_scatter(deg_v, [d], ones)
            return carry

        lax.fori_loop(0, E // 16, _deg_body, 0, unroll=4)

        def _dinv_body(i, carry):
            d = deg_v[pl.ds(i * 16, 16)] + 1.0
            bits = lax.bitcast_convert_type(d, jnp.int32)
            bits = jnp.int32(0x5F3759DF) - lax.shift_right_logical(bits, 1)
            y = lax.bitcast_convert_type(bits, jnp.float32)
            y = y * (1.5 - 0.5 * d * y * y)
            y = y * (1.5 - 0.5 * d * y * y)
            y = y * (1.5 - 0.5 * d * y * y)
            deg_v[pl.ds(i * 16, 16)] = y
            return carry

        lax.fori_loop(0, NP // 16, _dinv_body, 0, unroll=4)

        def _edge_body(e, carry):
            s = edge_v[0, pl.ds(e * 16, 16)]
            d = edge_v[1, pl.ds(e * 16, 16)]
            ns = plsc.load_gather(deg_v, [s])
            nd = plsc.load_gather(deg_v, [d])
            sl = s - lo
            mask = jnp.logical_and(sl >= 0, sl < BAND)
            sl = jnp.clip(sl, 0, BAND - 1)
            plsc.addupdate_scatter(band_v, [sl, d], ns * nd, mask=mask)
            return carry

        lax.fori_loop(0, E // 16, _edge_body, 0, unroll=4)

        def _diag_body(i, carry):
            n = lo + i * 16 + lax.iota(jnp.int32, 16)
            nl = n - lo
            mask = jnp.logical_and(nl < BAND, n < jnp.int32(N))
            nl = jnp.clip(nl, 0, BAND - 1)
            nc = lo + nl
            dv = plsc.load_gather(deg_v, [nc])
            plsc.addupdate_scatter(band_v, [nl, nc], dv * dv, mask=mask)
            return carry

        lax.fori_loop(0, (BAND + 15) // 16, _diag_body, 0)

        pltpu.sync_copy(band_v, at_ref.at[pl.ds(lo, BAND)])


@functools.cache
def _get_build_adj():
    return pl.kernel(
        _sc_build_adj_body,
        out_type=jax.ShapeDtypeStruct((NP, NP), jnp.float32),
        mesh=plsc.VectorSubcoreMesh(core_axis_name="c", subcore_axis_name="s"),
        scratch_types=[
            pltpu.VMEM((2, E), jnp.int32),
            pltpu.VMEM((BAND, NP), jnp.float32),
            pltpu.VMEM((NP,), jnp.float32),
        ],
        compiler_params=pltpu.CompilerParams(
            needs_layout_passes=False, skip_device_barrier=True
        ),
    )


def _tc_pre_body(wp_ref, wm_ref, bp_ref, bm_ref, sw_ref, q_ref, w0m_ref, brow_ref):
    wm = wm_ref[...]
    for cp in range(GC):
        wsum = wp_ref[1] * sw_ref[8 + cp * GC]
        for c in range(1, GC):
            wsum = wsum + wp_ref[c + 1] * sw_ref[8 + cp * GC + c]
        q_ref[cp] = jnp.dot(wsum, wm, preferred_element_type=jnp.float32)
    w0m_ref[...] = jnp.dot(wp_ref[0], wm, preferred_element_type=jnp.float32)
    bvec = bp_ref[...]
    for c in range(GC):
        bvec = bvec + sw_ref[24 + c] * jnp.sum(wp_ref[c + 1], axis=0, keepdims=True)
    brow_ref[...] = jnp.dot(bvec, wm, preferred_element_type=jnp.float32) + bm_ref[...]


def _tc_pre(wp5, wm, bp, bm, sw):
    return pl.pallas_call(
        _tc_pre_body,
        compiler_params=pltpu.CompilerParams(skip_device_barrier=True),
        in_specs=[
            pl.BlockSpec((1 + GC, NP, NP), lambda: (0, 0, 0)),
            pl.BlockSpec((NP, NP), lambda: (0, 0)),
            pl.BlockSpec((1, NP), lambda: (0, 0)),
            pl.BlockSpec((1, NP), lambda: (0, 0)),
            pl.BlockSpec(memory_space=pltpu.SMEM),
        ],
        out_specs=[
            pl.BlockSpec((GC, NP, NP), lambda: (0, 0, 0)),
            pl.BlockSpec((NP, NP), lambda: (0, 0)),
            pl.BlockSpec((1, NP), lambda: (0, 0)),
        ],
        out_shape=[
            jax.ShapeDtypeStruct((GC, NP, NP), jnp.float32),
            jax.ShapeDtypeStruct((NP, NP), jnp.float32),
            jax.ShapeDtypeStruct((1, NP), jnp.float32),
        ],
    )(wp5, wm, bp, bm, sw)


def _tc_main_body(x_ref, at_ref, q_ref, w0m_ref, brow_ref, sw_ref, out_ref, ab_ref, k_ref):
    i = pl.program_id(0)

    @pl.when(i == 0)
    def _():
        at = at_ref[...]
        ab_ref[:, :NP] = at.astype(jnp.bfloat16)
        ab_ref[:, NP:] = w0m_ref[...].astype(jnp.bfloat16)
        for c in range(GC):
            k_ref[c] = jnp.dot(
                at, q_ref[c], preferred_element_type=jnp.float32
            ).astype(jnp.bfloat16)

    x = x_ref[...]
    ga = jnp.dot(x, ab_ref[...], preferred_element_type=jnp.float32)
    g1 = ga[:, :NP]
    acc = ga[:, NP:] + brow_ref[...]
    for c in range(GC):
        h = jnp.maximum(g1 * sw_ref[c] + sw_ref[GC + c], 0.0)
        acc = acc + jnp.dot(
            h.astype(jnp.bfloat16), k_ref[c], preferred_element_type=jnp.float32
        )
    out_ref[...] = acc[:, :N].reshape(BM // 96, 96, N)


def _tc_main(xr, at, q, w0m, brow, sw):
    rows = xr.shape[0]
    return pl.pallas_call(
        _tc_main_body,
        compiler_params=pltpu.CompilerParams(skip_device_barrier=True),
        grid=(rows // BM,),
        in_specs=[
            pl.BlockSpec((BM, NP), lambda i: (i, 0)),
            pl.BlockSpec((NP, NP), lambda i: (0, 0)),
            pl.BlockSpec((GC, NP, NP), lambda i: (0, 0, 0)),
            pl.BlockSpec((NP, NP), lambda i: (0, 0)),
            pl.BlockSpec((1, NP), lambda i: (0, 0)),
            pl.BlockSpec(memory_space=pltpu.SMEM),
        ],
        out_specs=pl.BlockSpec((BM // 96, 96, N), lambda i: (i, 0, 0)),
        out_shape=jax.ShapeDtypeStruct((rows // 96, 96, N), jnp.float32),
        scratch_shapes=[
            pltpu.VMEM((NP, 2 * NP), jnp.bfloat16),
            pltpu.VMEM((GC, NP, NP), jnp.bfloat16),
        ],
    )(xr, at, q, w0m, brow, sw)


def kernel(x_enc, x_mark_enc, x_dec, x_mark_dec, edge_index, W1, b1, W2, b2, Wp, bp, Wm, bm):
    B_, T_, N_ = x_enc.shape
    xr = jnp.pad(x_enc.reshape(B_ * T_, N_), ((0, 0), (0, NP - N))).astype(jnp.bfloat16)
    at = _get_build_adj()(edge_index, jnp.zeros((NP, NP), jnp.float32))
    wp5 = jnp.pad(
        Wp.reshape(N, 1 + GC, N).transpose(1, 0, 2),
        ((0, 0), (0, NP - N), (0, NP - N)),
    )
    wm_p = jnp.pad(Wm, ((0, NP - N), (0, NP - N)))
    bp_p = jnp.pad(bp, (0, NP - N)).reshape(1, NP)
    bm_p = jnp.pad(bm, (0, NP - N)).reshape(1, NP)
    sw = jnp.concatenate([W1.reshape(-1), b1, W2.reshape(-1), b2])
    q, w0m, brow = _tc_pre(wp5, wm_p, bp_p, bm_p, sw)
    return _tc_main(xr, at, q, w0m, brow, sw)


# speedup vs baseline: 1.0009x; 1.0009x over previous
import functools

import jax
import jax.numpy as jnp
from jax import lax
from jax.experimental import pallas as pl
from jax.experimental.pallas import tpu as pltpu
from jax.experimental.pallas import tpu_sc as plsc

N = 325
E = 5200
GC = 4
NP = 384
BM = 384


BAND = NP // 16


def _sc_build_adj_body(edge_ref, zeros_ref, at_ref, edge_v, band_v, deg_v):
    cid = lax.axis_index("c")
    sid = lax.axis_index("s")

    @pl.when(cid == 0)
    def _():
        lo = sid * BAND
        pltpu.sync_copy(edge_ref, edge_v)
        pltpu.sync_copy(zeros_ref.at[pl.ds(0, BAND)], band_v)
        pltpu.sync_copy(zeros_ref.at[0], deg_v)

        ones = jnp.full((16,), 1.0, dtype=jnp.float32)

        def _deg_body(e, carry):
            d = edge_v[1, pl.ds(e * 16, 16)]
            plsc.addupdate_scatter(deg_v, [d], ones)
            return carry

        lax.fori_loop(0, E // 16, _deg_body, 0, unroll=4)

        def _dinv_body(i, carry):
            d = deg_v[pl.ds(i * 16, 16)] + 1.0
            bits = lax.bitcast_convert_type(d, jnp.int32)
            bits = jnp.int32(0x5F3759DF) - lax.shift_right_logical(bits, 1)
            y = lax.bitcast_convert_type(bits, jnp.float32)
            y = y * (1.5 - 0.5 * d * y * y)
            y = y * (1.5 - 0.5 * d * y * y)
            y = y * (1.5 - 0.5 * d * y * y)
            deg_v[pl.ds(i * 16, 16)] = y
            return carry

        lax.fori_loop(0, NP // 16, _dinv_body, 0, unroll=4)

        def _edge_body(e, carry):
            s = edge_v[0, pl.ds(e * 16, 16)]
            d = edge_v[1, pl.ds(e * 16, 16)]
            ns = plsc.load_gather(deg_v, [s])
            nd = plsc.load_gather(deg_v, [d])
            sl = s - lo
            mask = jnp.logical_and(sl >= 0, sl < BAND)
            sl = jnp.clip(sl, 0, BAND - 1)
            plsc.addupdate_scatter(band_v, [sl, d], ns * nd, mask=mask)
            return carry

        lax.fori_loop(0, E // 16, _edge_body, 0, unroll=4)

        def _diag_body(i, carry):
            n = lo + i * 16 + lax.iota(jnp.int32, 16)
            nl = n - lo
            mask = jnp.logical_and(nl < BAND, n < jnp.int32(N))
            nl = jnp.clip(nl, 0, BAND - 1)
            nc = lo + nl
            dv = plsc.load_gather(deg_v, [nc])
            plsc.addupdate_scatter(band_v, [nl, nc], dv * dv, mask=mask)
            return carry

        lax.fori_loop(0, (BAND + 15) // 16, _diag_body, 0)

        pltpu.sync_copy(band_v, at_ref.at[pl.ds(lo, BAND)])


@functools.cache
def _get_build_adj():
    return pl.kernel(
        _sc_build_adj_body,
        out_type=jax.ShapeDtypeStruct((NP, NP), jnp.float32),
        mesh=plsc.VectorSubcoreMesh(core_axis_name="c", subcore_axis_name="s"),
        scratch_types=[
            pltpu.VMEM((2, E), jnp.int32),
            pltpu.VMEM((BAND, NP), jnp.float32),
            pltpu.VMEM((NP,), jnp.float32),
        ],
        compiler_params=pltpu.CompilerParams(
            needs_layout_passes=False, skip_device_barrier=True
        ),
    )


def _tc_pre_body(wp_ref, wm_ref, bp_ref, bm_ref, sw_ref, q_ref, w0m_ref, brow_ref):
    wm = wm_ref[...].astype(jnp.bfloat16)
    for cp in range(GC):
        wsum = wp_ref[1] * sw_ref[8 + cp * GC]
        for c in range(1, GC):
            wsum = wsum + wp_ref[c + 1] * sw_ref[8 + cp * GC + c]
        q_ref[cp] = jnp.dot(
            wsum.astype(jnp.bfloat16), wm, preferred_element_type=jnp.float32
        )
    w0m_ref[...] = jnp.dot(
        wp_ref[0].astype(jnp.bfloat16), wm, preferred_element_type=jnp.float32
    )
    bvec = bp_ref[...]
    for c in range(GC):
        bvec = bvec + sw_ref[24 + c] * jnp.sum(wp_ref[c + 1], axis=0, keepdims=True)
    brow_ref[...] = jnp.dot(
        bvec.astype(jnp.bfloat16), wm, preferred_element_type=jnp.float32
    ) + bm_ref[...]


def _tc_pre(wp5, wm, bp, bm, sw):
    return pl.pallas_call(
        _tc_pre_body,
        compiler_params=pltpu.CompilerParams(skip_device_barrier=True),
        in_specs=[
            pl.BlockSpec((1 + GC, NP, NP), lambda: (0, 0, 0)),
            pl.BlockSpec((NP, NP), lambda: (0, 0)),
            pl.BlockSpec((1, NP), lambda: (0, 0)),
            pl.BlockSpec((1, NP), lambda: (0, 0)),
            pl.BlockSpec(memory_space=pltpu.SMEM),
        ],
        out_specs=[
            pl.BlockSpec((GC, NP, NP), lambda: (0, 0, 0)),
            pl.BlockSpec((NP, NP), lambda: (0, 0)),
            pl.BlockSpec((1, NP), lambda: (0, 0)),
        ],
        out_shape=[
            jax.ShapeDtypeStruct((GC, NP, NP), jnp.float32),
            jax.ShapeDtypeStruct((NP, NP), jnp.float32),
            jax.ShapeDtypeStruct((1, NP), jnp.float32),
        ],
    )(wp5, wm, bp, bm, sw)


def _tc_main_body(x_ref, at_ref, q_ref, w0m_ref, brow_ref, sw_ref, out_ref, ab_ref, k_ref):
    i = pl.program_id(0)

    @pl.when(i == 0)
    def _():
        at16 = at_ref[...].astype(jnp.bfloat16)
        ab_ref[:, :NP] = at16
        ab_ref[:, NP:] = w0m_ref[...].astype(jnp.bfloat16)
        for c in range(GC):
            k_ref[c] = jnp.dot(
                at16, q_ref[c].astype(jnp.bfloat16), preferred_element_type=jnp.float32
            ).astype(jnp.bfloat16)

    x = x_ref[...]
    ga = jnp.dot(x, ab_ref[...], preferred_element_type=jnp.float32)
    g1 = ga[:, :NP]
    acc = ga[:, NP:] + brow_ref[...]
    for c in range(GC):
        h = jnp.maximum(g1 * sw_ref[c] + sw_ref[GC + c], 0.0)
        acc = acc + jnp.dot(
            h.astype(jnp.bfloat16), k_ref[c], preferred_element_type=jnp.float32
        )
    out_ref[...] = acc[:, :N].reshape(BM // 96, 96, N)


def _tc_main(xr, at, q, w0m, brow, sw):
    rows = xr.shape[0]
    return pl.pallas_call(
        _tc_main_body,
        compiler_params=pltpu.CompilerParams(skip_device_barrier=True),
        grid=(rows // BM,),
        in_specs=[
            pl.BlockSpec((BM, NP), lambda i: (i, 0)),
            pl.BlockSpec((NP, NP), lambda i: (0, 0)),
            pl.BlockSpec((GC, NP, NP), lambda i: (0, 0, 0)),
            pl.BlockSpec((NP, NP), lambda i: (0, 0)),
            pl.BlockSpec((1, NP), lambda i: (0, 0)),
            pl.BlockSpec(memory_space=pltpu.SMEM),
        ],
        out_specs=pl.BlockSpec((BM // 96, 96, N), lambda i: (i, 0, 0)),
        out_shape=jax.ShapeDtypeStruct((rows // 96, 96, N), jnp.float32),
        scratch_shapes=[
            pltpu.VMEM((NP, 2 * NP), jnp.bfloat16),
            pltpu.VMEM((GC, NP, NP), jnp.bfloat16),
        ],
    )(xr, at, q, w0m, brow, sw)


def kernel(x_enc, x_mark_enc, x_dec, x_mark_dec, edge_index, W1, b1, W2, b2, Wp, bp, Wm, bm):
    B_, T_, N_ = x_enc.shape
    xr = jnp.pad(x_enc.reshape(B_ * T_, N_), ((0, 0), (0, NP - N))).astype(jnp.bfloat16)
    at = _get_build_adj()(edge_index, jnp.zeros((NP, NP), jnp.float32))
    wp5 = jnp.pad(
        Wp.reshape(N, 1 + GC, N).transpose(1, 0, 2),
        ((0, 0), (0, NP - N), (0, NP - N)),
    )
    wm_p = jnp.pad(Wm, ((0, NP - N), (0, NP - N)))
    bp_p = jnp.pad(bp, (0, NP - N)).reshape(1, NP)
    bm_p = jnp.pad(bm, (0, NP - N)).reshape(1, NP)
    sw = jnp.concatenate([W1.reshape(-1), b1, W2.reshape(-1), b2])
    q, w0m, brow = _tc_pre(wp5, wm_p, bp_p, bm_p, sw)
    return _tc_main(xr, at, q, w0m, brow, sw)


# speedup vs baseline: 1.0430x; 1.0420x over previous
import functools

import jax
import jax.numpy as jnp
from jax import lax
from jax.experimental import pallas as pl
from jax.experimental.pallas import tpu as pltpu
from jax.experimental.pallas import tpu_sc as plsc

N = 325
E = 5200
GC = 4
NP = 384
BM = 384


BAND = NP // 16


def _sc_build_adj_body(edge_ref, zeros_ref, at_ref, edge_v, band_v, deg_v):
    cid = lax.axis_index("c")
    sid = lax.axis_index("s")

    @pl.when(cid == 0)
    def _():
        lo = sid * BAND
        pltpu.sync_copy(edge_ref, edge_v)
        pltpu.sync_copy(zeros_ref.at[pl.ds(0, BAND)], band_v)
        pltpu.sync_copy(zeros_ref.at[0], deg_v)

        ones = jnp.full((16,), 1.0, dtype=jnp.float32)

        def _deg_body(e, carry):
            d = edge_v[1, pl.ds(e * 16, 16)]
            plsc.addupdate_scatter(deg_v, [d], ones)
            return carry

        lax.fori_loop(0, E // 16, _deg_body, 0, unroll=4)

        def _dinv_body(i, carry):
            d = deg_v[pl.ds(i * 16, 16)] + 1.0
            bits = lax.bitcast_convert_type(d, jnp.int32)
            bits = jnp.int32(0x5F3759DF) - lax.shift_right_logical(bits, 1)
            y = lax.bitcast_convert_type(bits, jnp.float32)
            y = y * (1.5 - 0.5 * d * y * y)
            y = y * (1.5 - 0.5 * d * y * y)
            y = y * (1.5 - 0.5 * d * y * y)
            deg_v[pl.ds(i * 16, 16)] = y
            return carry

        lax.fori_loop(0, NP // 16, _dinv_body, 0, unroll=4)

        def _edge_body(e, carry):
            s = edge_v[0, pl.ds(e * 16, 16)]
            d = edge_v[1, pl.ds(e * 16, 16)]
            ns = plsc.load_gather(deg_v, [s])
            nd = plsc.load_gather(deg_v, [d])
            sl = s - lo
            mask = jnp.logical_and(sl >= 0, sl < BAND)
            sl = jnp.clip(sl, 0, BAND - 1)
            plsc.addupdate_scatter(band_v, [sl, d], ns * nd, mask=mask)
            return carry

        lax.fori_loop(0, E // 16, _edge_body, 0, unroll=4)

        def _diag_body(i, carry):
            n = lo + i * 16 + lax.iota(jnp.int32, 16)
            nl = n - lo
            mask = jnp.logical_and(nl < BAND, n < jnp.int32(N))
            nl = jnp.clip(nl, 0, BAND - 1)
            nc = lo + nl
            dv = plsc.load_gather(deg_v, [nc])
            plsc.addupdate_scatter(band_v, [nl, nc], dv * dv, mask=mask)
            return carry

        lax.fori_loop(0, (BAND + 15) // 16, _diag_body, 0)

        pltpu.sync_copy(band_v, at_ref.at[pl.ds(lo, BAND)])


@functools.cache
def _get_build_adj():
    return pl.kernel(
        _sc_build_adj_body,
        out_type=jax.ShapeDtypeStruct((NP, NP), jnp.float32),
        mesh=plsc.VectorSubcoreMesh(core_axis_name="c", subcore_axis_name="s"),
        scratch_types=[
            pltpu.VMEM((2, E), jnp.int32),
            pltpu.VMEM((BAND, NP), jnp.float32),
            pltpu.VMEM((NP,), jnp.float32),
        ],
        compiler_params=pltpu.CompilerParams(
            needs_layout_passes=False, skip_device_barrier=True
        ),
    )


def _tc_main_body(
    x_ref, at_ref, wp_ref, wm_ref, bp_ref, bm_ref, sw_ref, out_ref, ab_ref, k_ref, brow_ref
):
    i = pl.program_id(0)

    @pl.when(i == 0)
    def _():
        at16 = at_ref[...].astype(jnp.bfloat16)
        wm16 = wm_ref[...].astype(jnp.bfloat16)
        ab_ref[:, :NP] = at16
        ab_ref[:, NP:] = jnp.dot(
            wp_ref[0].astype(jnp.bfloat16), wm16, preferred_element_type=jnp.float32
        ).astype(jnp.bfloat16)
        for cp in range(GC):
            wsum = wp_ref[1] * sw_ref[8 + cp * GC]
            for c in range(1, GC):
                wsum = wsum + wp_ref[c + 1] * sw_ref[8 + cp * GC + c]
            q = jnp.dot(
                wsum.astype(jnp.bfloat16), wm16, preferred_element_type=jnp.float32
            )
            k_ref[cp] = jnp.dot(
                at16, q.astype(jnp.bfloat16), preferred_element_type=jnp.float32
            ).astype(jnp.bfloat16)
        bvec = bp_ref[...]
        for c in range(GC):
            bvec = bvec + sw_ref[24 + c] * jnp.sum(wp_ref[c + 1], axis=0, keepdims=True)
        brow_ref[...] = jnp.dot(
            bvec.astype(jnp.bfloat16), wm16, preferred_element_type=jnp.float32
        ) + bm_ref[...]

    x = x_ref[...]
    ga = jnp.dot(x, ab_ref[...], preferred_element_type=jnp.float32)
    g1 = ga[:, :NP]
    acc = ga[:, NP:] + brow_ref[...]
    for c in range(GC):
        h = jnp.maximum(g1 * sw_ref[c] + sw_ref[GC + c], 0.0)
        acc = acc + jnp.dot(
            h.astype(jnp.bfloat16), k_ref[c], preferred_element_type=jnp.float32
        )
    out_ref[...] = acc[:, :N].reshape(BM // 96, 96, N)


def _tc_main(xr, at, wp5, wm, bp, bm, sw):
    rows = xr.shape[0]
    return pl.pallas_call(
        _tc_main_body,
        compiler_params=pltpu.CompilerParams(skip_device_barrier=True),
        grid=(rows // BM,),
        in_specs=[
            pl.BlockSpec((BM, NP), lambda i: (i, 0)),
            pl.BlockSpec((NP, NP), lambda i: (0, 0)),
            pl.BlockSpec((1 + GC, NP, NP), lambda i: (0, 0, 0)),
            pl.BlockSpec((NP, NP), lambda i: (0, 0)),
            pl.BlockSpec((1, NP), lambda i: (0, 0)),
            pl.BlockSpec((1, NP), lambda i: (0, 0)),
            pl.BlockSpec(memory_space=pltpu.SMEM),
        ],
        out_specs=pl.BlockSpec((BM // 96, 96, N), lambda i: (i, 0, 0)),
        out_shape=jax.ShapeDtypeStruct((rows // 96, 96, N), jnp.float32),
        scratch_shapes=[
            pltpu.VMEM((NP, 2 * NP), jnp.bfloat16),
            pltpu.VMEM((GC, NP, NP), jnp.bfloat16),
            pltpu.VMEM((1, NP), jnp.float32),
        ],
    )(xr, at, wp5, wm, bp, bm, sw)


def kernel(x_enc, x_mark_enc, x_dec, x_mark_dec, edge_index, W1, b1, W2, b2, Wp, bp, Wm, bm):
    B_, T_, N_ = x_enc.shape
    xr = jnp.pad(x_enc.reshape(B_ * T_, N_), ((0, 0), (0, NP - N))).astype(jnp.bfloat16)
    at = _get_build_adj()(edge_index, jnp.zeros((NP, NP), jnp.float32))
    wp5 = jnp.pad(
        Wp.reshape(N, 1 + GC, N).transpose(1, 0, 2),
        ((0, 0), (0, NP - N), (0, NP - N)),
    )
    wm_p = jnp.pad(Wm, ((0, NP - N), (0, NP - N)))
    bp_p = jnp.pad(bp, (0, NP - N)).reshape(1, NP)
    bm_p = jnp.pad(bm, (0, NP - N)).reshape(1, NP)
    sw = jnp.concatenate([W1.reshape(-1), b1, W2.reshape(-1), b2])
    return _tc_main(xr, at, wp5, wm_p, bp_p, bm_p, sw)
